# baseline (speedup 1.0000x reference)
import functools

import jax
import jax.numpy as jnp
import numpy as np
from jax import lax
from jax.experimental import pallas as pl
from jax.experimental.pallas import tpu as pltpu
from jax.experimental.pallas import tpu_sc as plsc

VOCAB = 1000000
EMBED_DIM = 64
NOISE_ALPHA = 5.0


_NC, _NS = 2, 16
_NW = _NC * _NS


def _sc_gather_body(batch, seq, table_hbm, x1_hbm, x2_hbm, out_hbm,
                    idx_v, rows_v, si0, si1, sg0, sg1, ss0, ss1):
    wid = lax.axis_index("s") * _NC + lax.axis_index("c")
    xrows_per_w = batch // _NW
    r0 = wid * xrows_per_w
    n16 = seq // 16
    tail = seq - n16 * 16
    sem_i = (si0, si1)
    sem_g = (sg0, sg1)
    sem_s = (ss0, ss1)

    def issue_idx(r, b):
        pltpu.async_copy(x1_hbm.at[r], idx_v.at[b, pl.ds(0, 128)], sem_i[b])
        pltpu.async_copy(x2_hbm.at[r], idx_v.at[b, pl.ds(128, 128)], sem_i[b])

    def wait_idx(b):
        pltpu.make_async_copy(x1_hbm.at[0],
                              idx_v.at[b, pl.ds(0, 128)], sem_i[b]).wait()
        pltpu.make_async_copy(x2_hbm.at[0],
                              idx_v.at[b, pl.ds(128, 128)], sem_i[b]).wait()

    issue_idx(r0, 0)

    def step(j, carry):
        for b in (0, 1):
            r = r0 + 2 * j + b

            @pl.when(r + 1 < r0 + xrows_per_w)
            def _():
                issue_idx(r + 1, 1 - b)

            wait_idx(b)

            @pl.when(r >= r0 + 2)
            def _():
                pltpu.make_async_copy(
                    table_hbm.at[pl.ds(0, seq)], rows_v.at[b],
                    sem_s[b]).wait()

            def fire16(k, c):
                v = idx_v[b, pl.ds(k * 16, 16)]
                for t in range(16):
                    pltpu.async_copy(table_hbm.at[pl.ds(v[t], 1)],
                                     rows_v.at[b, pl.ds(k * 16 + t, 1)],
                                     sem_g[b])
                return c

            lax.fori_loop(0, n16, fire16, 0, unroll=False)
            if tail:
                v = idx_v[b, pl.ds(n16 * 16, 16)]
                for t in range(tail):
                    pltpu.async_copy(table_hbm.at[pl.ds(v[t], 1)],
                                     rows_v.at[b, pl.ds(n16 * 16 + t, 1)],
                                     sem_g[b])

            @pl.when(r >= r0 + 1)
            def _():
                pltpu.make_async_copy(table_hbm.at[pl.ds(0, seq)],
                                      rows_v.at[1 - b], sem_g[1 - b]).wait()
                pltpu.async_copy(rows_v.at[1 - b],
                                 out_hbm.at[pl.ds((r - 1) * seq, seq)],
                                 sem_s[1 - b])
        return carry

    lax.fori_loop(0, xrows_per_w // 2, step, 0, unroll=False)

    last = r0 + xrows_per_w - 1
    pltpu.make_async_copy(table_hbm.at[pl.ds(0, seq)], rows_v.at[1],
                          sem_g[1]).wait()
    pltpu.sync_copy(rows_v.at[1], out_hbm.at[pl.ds(last * seq, seq)])
    pltpu.make_async_copy(table_hbm.at[pl.ds(0, seq)], rows_v.at[0],
                          sem_s[0]).wait()


def _sc_gather(table, x1, x2, batch, seq):
    nrows = batch * seq
    mesh = plsc.VectorSubcoreMesh(core_axis_name="c", subcore_axis_name="s")
    return pl.kernel(
        functools.partial(_sc_gather_body, batch, seq),
        out_type=jax.ShapeDtypeStruct((nrows, EMBED_DIM), jnp.float32),
        mesh=mesh,
        compiler_params=pltpu.CompilerParams(use_tc_tiling_on_sc=True),
        scratch_types=[
            pltpu.VMEM((2, 256), jnp.int32),
            pltpu.VMEM((2, seq, EMBED_DIM), jnp.float32),
            pltpu.SemaphoreType.DMA,
            pltpu.SemaphoreType.DMA,
            pltpu.SemaphoreType.DMA,
            pltpu.SemaphoreType.DMA,
            pltpu.SemaphoreType.DMA,
            pltpu.SemaphoreType.DMA,
        ],
    )(table, x1, x2)


_ROT_A = (13, 15, 26, 6)
_ROT_B = (17, 29, 16, 24)


def _threefry_noise(lo, mag):
    u32 = jnp.uint32
    ks0 = u32(0)
    ks1 = u32(42)
    ks2 = ks0 ^ ks1 ^ u32(0x1BD11BDA)
    ks = (ks0, ks1, ks2)
    x0 = jnp.zeros_like(lo)
    x1 = lo + ks1

    def rotl(v, d):
        return (v << u32(d)) | (v >> u32(32 - d))

    for i in range(5):
        rots = _ROT_A if i % 2 == 0 else _ROT_B
        for r in rots:
            x0 = x0 + x1
            x1 = rotl(x1, r)
            x1 = x1 ^ x0
        x0 = x0 + ks[(i + 1) % 3]
        x1 = x1 + ks[(i + 2) % 3] + u32(i + 1)

    bits = x0 ^ x1
    fl = lax.bitcast_convert_type((bits >> u32(9)) | u32(0x3F800000),
                                  jnp.float32) - jnp.float32(1.0)
    return jnp.maximum(jnp.float32(-mag),
                       fl * jnp.float32(2.0 * mag) + jnp.float32(-mag))


def _noise_body(mag, blk_l, d, blk_b, l, g_ref, out_ref):
    u32 = jnp.uint32
    shape = (blk_l, d, blk_b)
    i = pl.program_id(0)
    j = pl.program_id(1)
    li = lax.broadcasted_iota(u32, shape, 0) + (i * blk_l).astype(u32)
    di = lax.broadcasted_iota(u32, shape, 1)
    bi = lax.broadcasted_iota(u32, shape, 2) + (j * blk_b).astype(u32)
    lo = bi * u32(l * d) + li * u32(d) + di
    out_ref[...] = g_ref[...] + _threefry_noise(lo, mag)


def _tc_noise_add(g_t, mag, b, l):
    blk_l, blk_b = 25, 512
    grid = (l // blk_l, b // blk_b)
    spec = pl.BlockSpec((blk_l, EMBED_DIM, blk_b), lambda i, j: (i, 0, j))
    return pl.pallas_call(
        functools.partial(_noise_body, mag, blk_l, EMBED_DIM, blk_b, l),
        out_shape=jax.ShapeDtypeStruct((l, EMBED_DIM, b), jnp.float32),
        grid=grid,
        in_specs=[spec],
        out_specs=spec,
    )(g_t)


def kernel(x, table):
    b, l = x.shape
    x1 = x[:, :128]
    x2 = jnp.pad(x[:, 128:], ((0, 0), (0, 256 - l)))

    dims = np.float32(l * EMBED_DIM)
    mag = np.float32(NOISE_ALPHA) / np.sqrt(dims)

    gathered = _sc_gather(table, x1, x2, b, l)
    g_t = jnp.transpose(gathered.reshape(b, l, EMBED_DIM), (1, 2, 0))
    out_t = _tc_noise_add(g_t, mag, b, l)
    return jnp.transpose(out_t, (2, 0, 1))
